# baseline (speedup 1.0000x reference)
import functools

import numpy as np

import jax
import jax.numpy as jnp
from jax import lax
from jax.experimental import pallas as pl
from jax.experimental.pallas import tpu as pltpu
from jax.experimental.pallas import tpu_sc as plsc

N = 10000
E = 320000
D = 128

NC = 2
NS = 16
NW = NC * NS
C = 128
CHUNKS_A = 40
K_MAX = NW * C * CHUNKS_A
TREP = 256
TRASH = N
N_ACC = 10240
RPS = N_ACC // NS
R = 2048
G = N_ACC // R


def _sc_gsa_body(chunks, off, y_hbm, idx_hbm, zf_hbm, out_hbm,
                 acc, i0, i1, buf0, buf1, sem0, sem1):
    c = lax.axis_index("c")
    s = lax.axis_index("s")
    wid = s * NC + c
    pltpu.sync_copy(zf_hbm, acc.at[pl.ds(s * RPS, RPS)])
    plsc.subcore_barrier()
    base = off + wid * chunks * 2

    def ld(jj, ib):
        pltpu.sync_copy(idx_hbm.at[pl.ds(base + jj * 2, 2)], ib)

    ld(0, i0)
    pltpu.async_copy(y_hbm.at[i0.at[0]], buf0, sem0)

    def body(i, carry):
        j = 2 * i
        ld(j + 1, i1)
        pltpu.async_copy(y_hbm.at[i1.at[0]], buf1, sem1)
        pltpu.make_async_copy(y_hbm.at[i0.at[0]], buf0, sem0).wait()
        pltpu.sync_copy(buf0, acc.at[i0.at[1]], add=True)

        @pl.when(j + 2 < chunks)
        def _():
            ld(j + 2, i0)
            pltpu.async_copy(y_hbm.at[i0.at[0]], buf0, sem0)

        pltpu.make_async_copy(y_hbm.at[i1.at[0]], buf1, sem1).wait()
        pltpu.sync_copy(buf1, acc.at[i1.at[1]], add=True)
        return carry

    lax.fori_loop(0, chunks // 2, body, 0)
    plsc.subcore_barrier()
    pltpu.sync_copy(acc.at[pl.ds(s * RPS, RPS)],
                    out_hbm.at[pl.ds(c * N_ACC + s * RPS, RPS)])


@functools.cache
def _sc_make(chunks, off):
    mesh = plsc.VectorSubcoreMesh(core_axis_name="c", subcore_axis_name="s")
    return pl.kernel(
        functools.partial(_sc_gsa_body, chunks, off),
        out_type=jax.ShapeDtypeStruct((NC * N_ACC, D), jnp.float32),
        mesh=mesh,
        scratch_types=[
            pltpu.VMEM_SHARED((N_ACC, D), jnp.float32),
            pltpu.VMEM((2, C), jnp.int32),
            pltpu.VMEM((2, C), jnp.int32),
            pltpu.VMEM((C, D), jnp.float32),
            pltpu.VMEM((C, D), jnp.float32),
            pltpu.SemaphoreType.DMA,
            pltpu.SemaphoreType.DMA,
        ],
    )


def _dinv(g0_ref, g1_ref, off):
    d = g0_ref[:, off:off + 1] + g1_ref[:, off:off + 1] + 1.0
    return lax.rsqrt(d)


def _tc1_body(x_ref, w_ref, g0_ref, g1_ref, y_ref):
    xw = jnp.dot(x_ref[...], w_ref[...], preferred_element_type=jnp.float32)
    y_ref[...] = _dinv(g0_ref, g1_ref, 0) * xw


def _tc2_body(a0_ref, a1_ref, y1_ref, g0_ref, g1_ref, b1_ref, w2_ref, y2_ref):
    h = jnp.maximum(
        _dinv(g0_ref, g1_ref, 0) * (a0_ref[...] + a1_ref[...] + y1_ref[...])
        + b1_ref[...], 0.0)
    xw2 = jnp.dot(h, w2_ref[...], preferred_element_type=jnp.float32)
    y2_ref[...] = _dinv(g0_ref, g1_ref, 64) * xw2


def _tc3_body(q0_ref, q1_ref, y2_ref, g0_ref, g1_ref, b2_ref, wl_ref,
              bl_ref, o_ref):
    h2 = jnp.maximum(
        _dinv(g0_ref, g1_ref, 64) * (q0_ref[...] + q1_ref[...] + y2_ref[...])
        + b2_ref[...], 0.0)
    o_ref[...] = jnp.dot(h2, wl_ref[...],
                         preferred_element_type=jnp.float32) + bl_ref[...]


def _rows(i):
    return (i, 0)


def _at(blk):
    return lambda i: (blk + i, 0)


def _fixed(i):
    return (0, 0)


_spec_feat = pl.BlockSpec((R, D), _rows)
_spec_w = pl.BlockSpec((D, D), _fixed)
_spec_b = pl.BlockSpec((1, D), _fixed)


def _spec_part(blk):
    return pl.BlockSpec((R, D), _at(blk))


_tc1 = pl.pallas_call(
    _tc1_body,
    grid=(G,),
    in_specs=[_spec_feat, _spec_w, _spec_part(0), _spec_part(G)],
    out_specs=_spec_feat,
    out_shape=jax.ShapeDtypeStruct((N_ACC, D), jnp.float32),
)

_tc2 = pl.pallas_call(
    _tc2_body,
    grid=(G,),
    in_specs=[_spec_part(0), _spec_part(G), _spec_feat,
              _spec_part(0), _spec_part(G), _spec_b, _spec_w],
    out_specs=_spec_feat,
    out_shape=jax.ShapeDtypeStruct((N_ACC, D), jnp.float32),
)

_tc3 = pl.pallas_call(
    _tc3_body,
    grid=(G,),
    in_specs=[_spec_part(0), _spec_part(G), _spec_feat,
              _spec_part(0), _spec_part(G), _spec_b, _spec_w, _spec_b],
    out_specs=_spec_feat,
    out_shape=jax.ShapeDtypeStruct((N_ACC, D), jnp.float32),
)


@functools.cache
def _edge_constants():
    with jax.ensure_compile_time_eval():
        dk = jax.random.key(42)
        ka, kb = jax.random.split(dk)
        keep1 = np.asarray(jax.random.bernoulli(ka, 0.5, (E,)))
        keep2 = np.asarray(jax.random.bernoulli(kb, 0.5, (E,)))
    perm1 = np.argsort(~keep1, kind="stable")[:K_MAX].astype(np.int64)
    perm2 = np.argsort(~keep2, kind="stable")[:K_MAX].astype(np.int64)
    keepd = keep1 | keep2
    chd = 2 * ((int(keepd.sum()) + 2 * NW * C - 1) // (2 * NW * C))
    e_d = chd * NW * C
    permd = np.argsort(~keepd, kind="stable")[:e_d].astype(np.int64)
    kpat = keep1.astype(np.int32) + 2 * keep2.astype(np.int32)
    kgd = (kpat[permd] * TREP
           + np.arange(e_d, dtype=np.int32) % TREP).astype(np.int32)

    def interleave(g, sc):
        return np.stack([g.reshape(-1, C), sc.reshape(-1, C)],
                        axis=1).reshape(-1, C)

    idxall = np.concatenate([
        interleave(4 * E + np.arange(e_d, dtype=np.int64), 3 * E + permd),
        interleave(perm1, E + perm1),
        interleave(perm2, 2 * E + perm2),
    ]).astype(np.int32)
    return (jnp.asarray(keep1), jnp.asarray(keep2), jnp.asarray(kgd),
            jnp.asarray(idxall), chd)


def kernel(x, edge_index, W1, b1, W2, b2, Wlin, blin):
    keep1, keep2, kgd, idxall, chd = _edge_constants()
    nd = chd * NW
    n1 = K_MAX // C

    row, col = edge_index[0], edge_index[1]
    colt1 = jnp.where(keep1, col, TRASH)
    colt2 = jnp.where(keep2, col, TRASH)
    gsrc = jnp.concatenate([row, colt1, colt2, col, kgd])
    allidx = gsrc[idxall]

    patterns = np.zeros((4, D), np.float32)
    patterns[1, :64] = 1.0
    patterns[2, 64:] = 1.0
    patterns[3] = 1.0
    table = jnp.asarray(np.repeat(patterns, TREP, axis=0))
    zf = jnp.zeros((RPS, D), jnp.float32)
    xp = jnp.zeros((N_ACC, D), jnp.float32).at[:N].set(x)

    _sc_deg = _sc_make(chd, 0)
    _sc_agg1 = _sc_make(CHUNKS_A, 2 * nd)
    _sc_agg2 = _sc_make(CHUNKS_A, 2 * (nd + n1))
    degq = _sc_deg(table, allidx, zf)
    y1 = _tc1(xp, W1, degq, degq)
    a = _sc_agg1(y1, allidx, zf)
    y2 = _tc2(a, a, y1, degq, degq, b1.reshape(1, D), W2)
    q = _sc_agg2(y2, allidx, zf)
    out = _tc3(q, q, y2, degq, degq, b2.reshape(1, D), Wlin, blin.reshape(1, D))
    return out[:N]


# speedup vs baseline: 1.0963x; 1.0963x over previous
import functools

import numpy as np

import jax
import jax.numpy as jnp
from jax import lax
from jax.experimental import pallas as pl
from jax.experimental.pallas import tpu as pltpu
from jax.experimental.pallas import tpu_sc as plsc

N = 10000
E = 320000
D = 128

NC = 2
NS = 16
NW = NC * NS
C = 128
CHUNKS_A = 40
K_MAX = NW * C * CHUNKS_A
TREP = 256
TRASH = N
N_ACC = 10240
RPS = N_ACC // NS
R = 2048
G = N_ACC // R


def _sc_gsa_body(chunks, y_hbm, idx_hbm, zf_hbm, out_hbm,
                 acc, i0, i1, buf0, buf1, sem0, sem1):
    c = lax.axis_index("c")
    s = lax.axis_index("s")
    wid = s * NC + c
    pltpu.sync_copy(zf_hbm, acc.at[pl.ds(s * RPS, RPS)])
    plsc.subcore_barrier()
    base = wid * chunks * 2

    def ld(jj, ib):
        pltpu.sync_copy(idx_hbm.at[pl.ds(base + jj * 2, 2)], ib)

    ld(0, i0)
    pltpu.async_copy(y_hbm.at[i0.at[0]], buf0, sem0)

    def body(i, carry):
        j = 2 * i
        ld(j + 1, i1)
        pltpu.async_copy(y_hbm.at[i1.at[0]], buf1, sem1)
        pltpu.make_async_copy(y_hbm.at[i0.at[0]], buf0, sem0).wait()
        pltpu.sync_copy(buf0, acc.at[i0.at[1]], add=True)

        @pl.when(j + 2 < chunks)
        def _():
            ld(j + 2, i0)
            pltpu.async_copy(y_hbm.at[i0.at[0]], buf0, sem0)

        pltpu.make_async_copy(y_hbm.at[i1.at[0]], buf1, sem1).wait()
        pltpu.sync_copy(buf1, acc.at[i1.at[1]], add=True)
        return carry

    lax.fori_loop(0, chunks // 2, body, 0)
    plsc.subcore_barrier()
    pltpu.sync_copy(acc.at[pl.ds(s * RPS, RPS)],
                    out_hbm.at[pl.ds(c * N_ACC + s * RPS, RPS)])


@functools.cache
def _sc_make(chunks):
    mesh = plsc.VectorSubcoreMesh(core_axis_name="c", subcore_axis_name="s")
    return pl.kernel(
        functools.partial(_sc_gsa_body, chunks),
        out_type=jax.ShapeDtypeStruct((NC * N_ACC, D), jnp.float32),
        mesh=mesh,
        scratch_types=[
            pltpu.VMEM_SHARED((N_ACC, D), jnp.float32),
            pltpu.VMEM((2, C), jnp.int32),
            pltpu.VMEM((2, C), jnp.int32),
            pltpu.VMEM((C, D), jnp.float32),
            pltpu.VMEM((C, D), jnp.float32),
            pltpu.SemaphoreType.DMA,
            pltpu.SemaphoreType.DMA,
        ],
    )


def _dinv(g0_ref, g1_ref, off):
    d = g0_ref[:, off:off + 1] + g1_ref[:, off:off + 1] + 1.0
    return lax.rsqrt(d)


def _tc1_body(x_ref, w_ref, g0_ref, g1_ref, y_ref):
    xw = jnp.dot(x_ref[...], w_ref[...], preferred_element_type=jnp.float32)
    y_ref[...] = _dinv(g0_ref, g1_ref, 0) * xw


def _tc2_body(a0_ref, a1_ref, y1_ref, g0_ref, g1_ref, b1_ref, w2_ref, y2_ref):
    h = jnp.maximum(
        _dinv(g0_ref, g1_ref, 0) * (a0_ref[...] + a1_ref[...] + y1_ref[...])
        + b1_ref[...], 0.0)
    xw2 = jnp.dot(h, w2_ref[...], preferred_element_type=jnp.float32)
    y2_ref[...] = _dinv(g0_ref, g1_ref, 64) * xw2


def _tc3_body(q0_ref, q1_ref, y2_ref, g0_ref, g1_ref, b2_ref, wl_ref,
              bl_ref, o_ref):
    h2 = jnp.maximum(
        _dinv(g0_ref, g1_ref, 64) * (q0_ref[...] + q1_ref[...] + y2_ref[...])
        + b2_ref[...], 0.0)
    o_ref[...] = jnp.dot(h2, wl_ref[...],
                         preferred_element_type=jnp.float32) + bl_ref[...]


def _rows(i):
    return (i, 0)


def _at(blk):
    return lambda i: (blk + i, 0)


def _fixed(i):
    return (0, 0)


_spec_feat = pl.BlockSpec((R, D), _rows)
_spec_w = pl.BlockSpec((D, D), _fixed)
_spec_b = pl.BlockSpec((1, D), _fixed)


def _spec_part(blk):
    return pl.BlockSpec((R, D), _at(blk))


_tc1 = pl.pallas_call(
    _tc1_body,
    grid=(G,),
    in_specs=[_spec_feat, _spec_w, _spec_part(0), _spec_part(G)],
    out_specs=_spec_feat,
    out_shape=jax.ShapeDtypeStruct((N_ACC, D), jnp.float32),
)

_tc2 = pl.pallas_call(
    _tc2_body,
    grid=(G,),
    in_specs=[_spec_part(0), _spec_part(G), _spec_feat,
              _spec_part(0), _spec_part(G), _spec_b, _spec_w],
    out_specs=_spec_feat,
    out_shape=jax.ShapeDtypeStruct((N_ACC, D), jnp.float32),
)

_tc3 = pl.pallas_call(
    _tc3_body,
    grid=(G,),
    in_specs=[_spec_part(0), _spec_part(G), _spec_feat,
              _spec_part(0), _spec_part(G), _spec_b, _spec_w, _spec_b],
    out_specs=_spec_feat,
    out_shape=jax.ShapeDtypeStruct((N_ACC, D), jnp.float32),
)


@functools.cache
def _edge_constants():
    with jax.ensure_compile_time_eval():
        dk = jax.random.key(42)
        ka, kb = jax.random.split(dk)
        keep1 = np.asarray(jax.random.bernoulli(ka, 0.5, (E,)))
        keep2 = np.asarray(jax.random.bernoulli(kb, 0.5, (E,)))
    perm1 = np.argsort(~keep1, kind="stable")[:K_MAX].astype(np.int32)
    perm2 = np.argsort(~keep2, kind="stable")[:K_MAX].astype(np.int32)
    keepd = keep1 | keep2
    chd = 2 * ((int(keepd.sum()) + 2 * NW * C - 1) // (2 * NW * C))
    e_d = chd * NW * C
    permd = np.argsort(~keepd, kind="stable")[:e_d].astype(np.int32)
    kpat = keep1.astype(np.int32) + 2 * keep2.astype(np.int32)
    kgd = (kpat[permd] * TREP
           + np.arange(e_d, dtype=np.int32) % TREP).astype(np.int32)
    return (jnp.asarray(keep1), jnp.asarray(keep2), jnp.asarray(perm1),
            jnp.asarray(perm2), jnp.asarray(permd), jnp.asarray(kgd), chd)


def kernel(x, edge_index, W1, b1, W2, b2, Wlin, blin):
    keep1, keep2, perm1, perm2, permd, kgd, chd = _edge_constants()

    row, col = edge_index[0], edge_index[1]

    def combined(g, sc):
        n = g.shape[0] // C
        return jnp.stack(
            [g.reshape(n, C), sc.reshape(n, C)], axis=1).reshape(2 * n, C)

    idx1 = combined(row[perm1], jnp.where(keep1, col, TRASH)[perm1])
    idx2 = combined(row[perm2], jnp.where(keep2, col, TRASH)[perm2])
    idxd = combined(kgd, col[permd])

    patterns = np.zeros((4, D), np.float32)
    patterns[1, :64] = 1.0
    patterns[2, 64:] = 1.0
    patterns[3] = 1.0
    table = jnp.asarray(np.repeat(patterns, TREP, axis=0))
    zf = jnp.zeros((RPS, D), jnp.float32)
    xp = jnp.zeros((N_ACC, D), jnp.float32).at[:N].set(x)

    _sc_deg, _sc_agg = _sc_make(chd), _sc_make(CHUNKS_A)
    degq = _sc_deg(table, idxd, zf)
    y1 = _tc1(xp, W1, degq, degq)
    a = _sc_agg(y1, idx1, zf)
    y2 = _tc2(a, a, y1, degq, degq, b1.reshape(1, D), W2)
    q = _sc_agg(y2, idx2, zf)
    out = _tc3(q, q, y2, degq, degq, b2.reshape(1, D), Wlin, blin.reshape(1, D))
    return out[:N]
